# Initial kernel scaffold; baseline (speedup 1.0000x reference)
import jax
import jax.numpy as jnp
from jax.experimental import pallas as pl


def kernel(x, edge_index, ws0, bs0, wd0, bd0, a0, wres0, g0, be0, wsr, bsr, wdr, bdr, ar, gr, ber, w1, b1, gf, bf, w2, b2):
    raise NotImplementedError("write your pallas kernel here")


# speedup vs baseline: 1.0056x; 1.0056x over previous
import functools

import jax
import jax.numpy as jnp
from jax import lax
from jax.experimental import pallas as pl
from jax.experimental.pallas import tpu as pltpu

N = 50000
E = 800000
B = 50
NODES = 64
HEADS = 1
DH = NODES // HEADS
IN0 = 21
LAYERS = 4


def _head_body(feats_ref, w1_ref, b1_ref, gf_ref, bf_ref, w2_ref, b2_ref,
               out_ref):
    feats = feats_ref[...]
    f1 = jnp.dot(feats, w1_ref[...], preferred_element_type=jnp.float32)
    f1 = f1 + b1_ref[...][None, :]
    mu = jnp.mean(f1, axis=0, keepdims=True)
    var = jnp.mean((f1 - mu) ** 2, axis=0, keepdims=True)
    f1 = (f1 - mu) * lax.rsqrt(var + 1e-5) * gf_ref[...][None, :]
    f1 = f1 + bf_ref[...][None, :]
    f1 = jnp.maximum(f1, 0.0)
    out = jnp.dot(f1, w2_ref[...], preferred_element_type=jnp.float32)
    out_ref[...] = out + b2_ref[...][None, :]


def _head(feats, w1, b1, gf, bf, w2, b2):
    return pl.pallas_call(
        _head_body,
        out_shape=jax.ShapeDtypeStruct((B, 5), jnp.float32),
    )(feats, w1, b1, gf, bf, w2, b2)


def _gat_layer(h, src, dst, Ws, bs, Wd, bd, a, gamma, beta, Wres):
    n = h.shape[0]
    fs = (h @ Ws + bs).reshape(n, HEADS, DH)
    fd = (h @ Wd + bd).reshape(n, HEADS, DH)
    e = jax.nn.leaky_relu(fs[src] + fd[dst], 0.2)
    logits = jnp.sum(e * a[None, :, :], axis=-1)
    m = jax.ops.segment_max(logits, dst, num_segments=n)
    m = jnp.where(jnp.isfinite(m), m, 0.0)
    ex = jnp.exp(logits - m[dst])
    den = jax.ops.segment_sum(ex, dst, num_segments=n)
    alpha = ex / (den[dst] + 1e-9)
    out = jax.ops.segment_sum(fs[src] * alpha[:, :, None], dst, num_segments=n)
    res = h if Wres is None else h @ Wres
    out = out.reshape(n, NODES) + res
    mu = jnp.mean(out, axis=0)
    var = jnp.var(out, axis=0)
    out = (out - mu) / jnp.sqrt(var + 1e-5) * gamma + beta
    return jax.nn.relu(out), alpha


def kernel(x, edge_index, ws0, bs0, wd0, bd0, a0, wres0, g0, be0, wsr, bsr,
           wdr, bdr, ar, gr, ber, w1, b1, gf, bf, w2, b2):
    src = edge_index[0]
    dst = edge_index[1]
    gid = jnp.arange(N) // (N // B)
    npg = float(N // B)
    feats = []
    attns = []
    h, al = _gat_layer(x, src, dst, ws0, bs0, wd0, bd0, a0, g0, be0, wres0)
    feats.append(jax.ops.segment_sum(h, gid, num_segments=B) / npg)
    attns.append(al)
    for i in range(LAYERS - 1):
        h, al = _gat_layer(h, src, dst, wsr[i], bsr[i], wdr[i], bdr[i], ar[i],
                           gr[i], ber[i], None)
        feats.append(jax.ops.segment_sum(h, gid, num_segments=B) / npg)
        attns.append(al)
    feat = jnp.concatenate(feats, axis=1)
    feat = _head(feat, w1, b1, gf, bf, w2, b2)
    att = jnp.squeeze(jnp.stack(attns, axis=0)).T
    return feat, att


# speedup vs baseline: 4.8634x; 4.8364x over previous
import jax
import jax.numpy as jnp
from jax import lax
from jax.experimental import pallas as pl
from jax.experimental.pallas import tpu as pltpu
from jax.experimental.pallas import tpu_sc as plsc

N = 50000
E = 800000
B = 50
NODES = 64
HEADS = 1
DH = NODES // HEADS
IN0 = 21
LAYERS = 4

NC = 2
NS = 16
NW = NC * NS
L = 16

C = 128
NCH = E // C
NPAD = 51200
SL = NPAD // NS
HALF = 25000
OUTROWS = 25600
SPROWS = 25728
TRASH = OUTROWS
RB = 2000
GRID = N // RB

_f32 = jnp.float32
_i32 = jnp.int32

_SC_PARAMS = pltpu.CompilerParams(needs_layout_passes=False,
                                  use_tc_tiling_on_sc=False)


def _mesh():
    return plsc.VectorSubcoreMesh(core_axis_name="c", subcore_axis_name="s",
                                  num_cores=NC, num_subcores=NS)


def _wid():
    c = lax.axis_index("c")
    s = lax.axis_index("s")
    return c, s, s * NC + c


def _nchunks(w):
    return jnp.where(w < NCH % NW, NCH // NW + 1, NCH // NW)


def _ring_reduce(s, arr, accred, tmpred, msh, combine):
    def cp16(dst_ref, src_vals_ref, off):
        def body(j, _):
            dst_ref[pl.ds(j * L, L)] = src_vals_ref[pl.ds(off + j * L, L)]
            return _
        lax.fori_loop(0, SL // L, body, None)

    cp16(accred, arr, s * SL)
    for r in range(1, NS):
        send_slice = lax.rem(s + r, NS)
        pltpu.sync_copy(arr.at[pl.ds(send_slice * SL, SL)], msh.at[s])
        plsc.subcore_barrier()
        read_row = lax.rem(s - r + NS, NS)
        pltpu.sync_copy(msh.at[read_row], tmpred)

        def mx(j, _):
            sl = pl.ds(j * L, L)
            accred[sl] = combine(accred[sl], tmpred[sl])
            return _
        lax.fori_loop(0, SL // L, mx, None)
        plsc.subcore_barrier()


def _edge1_body(src, dst, fs, fd, avec, logits_o, mpart_o,
                sidx, didx, fsrows, fdrows, logbuf, tbuf, abuf_v,
                maxarr, accred, tmpred, msh, sem):
    c, s, w = _wid()

    def ini(j, _):
        maxarr[pl.ds(j * L, L)] = jnp.full((L,), -3.4e38, _f32)
        return _
    lax.fori_loop(0, NPAD // L, ini, None)

    pltpu.sync_copy(avec, abuf_v)
    aq = [abuf_v[pl.ds(q * L, L)] for q in range(4)]
    iot = lax.iota(_i32, L)

    def chunk(k, _):
        base = (w + k * NW) * C
        pltpu.sync_copy(src.at[pl.ds(base, C)], sidx)
        pltpu.sync_copy(dst.at[pl.ds(base, C)], didx)
        cp1 = pltpu.async_copy(fs.at[sidx], fsrows, sem)
        cp2 = pltpu.async_copy(fd.at[didx], fdrows, sem)
        cp1.wait()
        cp2.wait()

        def grp(g, _):
            def edge(i, _):
                e = g * L + i
                acc = jnp.zeros((L,), _f32)
                for q in range(4):
                    t = fsrows[e, pl.ds(q * L, L)] + fdrows[e, pl.ds(q * L, L)]
                    lr = 0.2 * t + 0.8 * jnp.maximum(t, 0.0)
                    acc = acc + lr * aq[q]
                tbuf[pl.ds(i * L, L)] = acc
                return _
            lax.fori_loop(0, L, edge, None)
            lg = jnp.zeros((L,), _f32)
            iot16 = iot * L
            for cc in range(L):
                lg = lg + plsc.load_gather(tbuf, [iot16 + cc])
            logbuf[pl.ds(g * L, L)] = lg
            dvec = didx[pl.ds(g * L, L)]
            old = plsc.load_gather(maxarr, [dvec])
            upd = lg > old
            plsc.store_scatter(maxarr, [dvec], jnp.maximum(old, lg), mask=upd)
            rb = plsc.load_gather(maxarr, [dvec])

            def w_body(p):
                o2 = plsc.load_gather(maxarr, [dvec])
                u2 = lg > o2
                plsc.store_scatter(maxarr, [dvec], jnp.maximum(o2, lg),
                                   mask=u2)
                r2 = plsc.load_gather(maxarr, [dvec])
                return jnp.any(r2 < lg)
            lax.while_loop(lambda p: p, w_body, jnp.any(rb < lg))
            return _
        lax.fori_loop(0, C // L, grp, None)
        pltpu.sync_copy(logbuf, logits_o.at[pl.ds(base, C)])
        return _
    lax.fori_loop(0, _nchunks(w), chunk, None)

    plsc.subcore_barrier()
    _ring_reduce(s, maxarr, accred, tmpred, msh, jnp.maximum)
    pltpu.sync_copy(accred, mpart_o.at[c, pl.ds(s * SL, SL)])


_edge1 = pl.kernel(
    _edge1_body,
    out_type=(jax.ShapeDtypeStruct((E,), _f32),
              jax.ShapeDtypeStruct((NC, NPAD), _f32)),
    mesh=_mesh(),
    compiler_params=_SC_PARAMS,
    scratch_types=[
        pltpu.VMEM((C,), _i32), pltpu.VMEM((C,), _i32),
        pltpu.VMEM((C, NODES), _f32), pltpu.VMEM((C, NODES), _f32),
        pltpu.VMEM((C,), _f32), pltpu.VMEM((L * L,), _f32),
        pltpu.VMEM((NODES,), _f32),
        pltpu.VMEM((NPAD,), _f32),
        pltpu.VMEM((SL,), _f32), pltpu.VMEM((SL,), _f32),
        pltpu.VMEM_SHARED((NS, SL), _f32),
        pltpu.SemaphoreType.DMA,
    ],
)


def _edge2_body(dst, logits, mpart, ex_o, denpart_o,
                didx, lbuf, exbuf, marr, denarr, accred, tmpred, msh):
    c, s, w = _wid()
    pltpu.sync_copy(mpart.at[0], marr)
    pltpu.sync_copy(mpart.at[1], denarr)

    def comb(j, _):
        sl = pl.ds(j * L, L)
        marr[sl] = jnp.maximum(marr[sl], denarr[sl])
        denarr[sl] = jnp.zeros((L,), _f32)
        return _
    lax.fori_loop(0, NPAD // L, comb, None)

    def chunk(k, _):
        base = (w + k * NW) * C
        pltpu.sync_copy(dst.at[pl.ds(base, C)], didx)
        pltpu.sync_copy(logits.at[pl.ds(base, C)], lbuf)

        def grp(g, _):
            dvec = didx[pl.ds(g * L, L)]
            lvec = lbuf[pl.ds(g * L, L)]
            mv = plsc.load_gather(marr, [dvec])
            exv = jnp.exp(lvec - mv)
            exbuf[pl.ds(g * L, L)] = exv
            plsc.addupdate_scatter(denarr, [dvec], exv)
            return _
        lax.fori_loop(0, C // L, grp, None)
        pltpu.sync_copy(exbuf, ex_o.at[pl.ds(base, C)])
        return _
    lax.fori_loop(0, _nchunks(w), chunk, None)

    plsc.subcore_barrier()
    _ring_reduce(s, denarr, accred, tmpred, msh, lambda a, b: a + b)
    pltpu.sync_copy(accred, denpart_o.at[c, pl.ds(s * SL, SL)])


_edge2 = pl.kernel(
    _edge2_body,
    out_type=(jax.ShapeDtypeStruct((E,), _f32),
              jax.ShapeDtypeStruct((NC, NPAD), _f32)),
    mesh=_mesh(),
    compiler_params=_SC_PARAMS,
    scratch_types=[
        pltpu.VMEM((C,), _i32), pltpu.VMEM((C,), _f32),
        pltpu.VMEM((C,), _f32),
        pltpu.VMEM((NPAD,), _f32), pltpu.VMEM((NPAD,), _f32),
        pltpu.VMEM((SL,), _f32), pltpu.VMEM((SL,), _f32),
        pltpu.VMEM_SHARED((NS, SL), _f32),
    ],
)


def _alpha_body(dst, ex, denpart, att_o,
                didx, exbuf, abuf, denarr, tmpd):
    c, s, w = _wid()
    pltpu.sync_copy(denpart.at[0], denarr)

    def comb_j(j, _):
        pltpu.sync_copy(denpart.at[1, pl.ds(j * SL, SL)], tmpd)

        def ad(t, _):
            denarr[pl.ds(j * SL + t * L, L)] = (
                denarr[pl.ds(j * SL + t * L, L)] + tmpd[pl.ds(t * L, L)])
            return _
        lax.fori_loop(0, SL // L, ad, None)
        return _
    lax.fori_loop(0, NS, comb_j, None)

    def chunk(k, _):
        base = (w + k * NW) * C
        pltpu.sync_copy(dst.at[pl.ds(base, C)], didx)
        pltpu.sync_copy(ex.at[pl.ds(base, C)], exbuf)

        def grp(g, _):
            dvec = didx[pl.ds(g * L, L)]
            exv = exbuf[pl.ds(g * L, L)]
            dv = plsc.load_gather(denarr, [dvec])
            abuf[pl.ds(g * L, L)] = exv / (dv + 1e-9)
            return _
        lax.fori_loop(0, C // L, grp, None)
        pltpu.sync_copy(abuf, att_o.at[pl.ds(base, C)])
        return _
    lax.fori_loop(0, _nchunks(w), chunk, None)


_alpha = pl.kernel(
    _alpha_body,
    out_type=jax.ShapeDtypeStruct((E,), _f32),
    mesh=_mesh(),
    compiler_params=_SC_PARAMS,
    scratch_types=[
        pltpu.VMEM((C,), _i32), pltpu.VMEM((C,), _f32),
        pltpu.VMEM((C,), _f32),
        pltpu.VMEM((NPAD,), _f32), pltpu.VMEM((SL,), _f32),
    ],
)


def _edge3_body(src, dst, att, fs, outp_o,
                sidx, didx, abuf, locidx, fsrows, rowbuf, spout, sem):
    c, s, w = _wid()

    def zrow(i, _):
        for q in range(4):
            rowbuf[i, pl.ds(q * L, L)] = jnp.zeros((L,), _f32)
        return _
    lax.fori_loop(0, C, zrow, None)

    nz = SPROWS // C
    zc = jnp.where(s < nz % NS, nz // NS + 1, nz // NS)

    def zch(k, _):
        blk = s + k * NS
        pltpu.sync_copy(rowbuf, spout.at[pl.ds(blk * C, C)])
        return _
    lax.fori_loop(0, zc, zch, None)
    plsc.subcore_barrier()

    lo = c * HALF

    def chunk(k, _):
        base = (s + k * NS) * C
        pltpu.sync_copy(src.at[pl.ds(base, C)], sidx)
        pltpu.sync_copy(dst.at[pl.ds(base, C)], didx)
        pltpu.sync_copy(att.at[pl.ds(base, C)], abuf)
        cp = pltpu.async_copy(fs.at[sidx], fsrows, sem)
        cp.wait()

        def grp(g, _):
            dvec = didx[pl.ds(g * L, L)]
            inh = (dvec >= lo) & (dvec < lo + HALF)
            locidx[pl.ds(g * L, L)] = jnp.where(inh, dvec - lo, TRASH)
            return _
        lax.fori_loop(0, C // L, grp, None)

        def edge(i, _):
            av = plsc.load_gather(abuf, [jnp.full((L,), i, _i32)])
            for q in range(4):
                rowbuf[i, pl.ds(q * L, L)] = fsrows[i, pl.ds(q * L, L)] * av
            return _
        lax.fori_loop(0, C, edge, None)
        pltpu.sync_copy(rowbuf, spout.at[locidx], add=True)
        return _
    nch = jnp.where(s < NCH % NS, NCH // NS + 1, NCH // NS)
    lax.fori_loop(0, nch, chunk, None)

    plsc.subcore_barrier()
    rows = OUTROWS // NS
    pltpu.sync_copy(spout.at[pl.ds(s * rows, rows)],
                    outp_o.at[c, pl.ds(s * rows, rows)])


_edge3 = pl.kernel(
    _edge3_body,
    out_type=jax.ShapeDtypeStruct((NC, OUTROWS, NODES), _f32),
    mesh=_mesh(),
    compiler_params=_SC_PARAMS,
    scratch_types=[
        pltpu.VMEM((C,), _i32), pltpu.VMEM((C,), _i32),
        pltpu.VMEM((C,), _f32), pltpu.VMEM((C,), _i32),
        pltpu.VMEM((C, NODES), _f32), pltpu.VMEM((C, NODES), _f32),
        pltpu.VMEM_SHARED((SPROWS, NODES), _f32),
        pltpu.SemaphoreType.DMA,
    ],
)


def _dense0_body(x_ref, ws_ref, bs_ref, wd_ref, bd_ref, wres_ref,
                 fs_o, fd_o, res_o):
    xb = x_ref[...]
    fs_o[...] = (jnp.dot(xb, ws_ref[...], preferred_element_type=_f32)
                 + bs_ref[...][None, :])
    fd_o[...] = (jnp.dot(xb, wd_ref[...], preferred_element_type=_f32)
                 + bd_ref[...][None, :])
    res_o[...] = jnp.dot(xb, wres_ref[...], preferred_element_type=_f32)


def _dense0(x, ws, bs, wd, bd, wres):
    out = jax.ShapeDtypeStruct((N, NODES), _f32)
    return pl.pallas_call(
        _dense0_body,
        grid=(GRID,),
        in_specs=[
            pl.BlockSpec((RB, IN0), lambda i: (i, 0)),
            pl.BlockSpec((IN0, NODES), lambda i: (0, 0)),
            pl.BlockSpec((NODES,), lambda i: (0,)),
            pl.BlockSpec((IN0, NODES), lambda i: (0, 0)),
            pl.BlockSpec((NODES,), lambda i: (0,)),
            pl.BlockSpec((IN0, NODES), lambda i: (0, 0)),
        ],
        out_specs=[pl.BlockSpec((RB, NODES), lambda i: (i, 0))] * 3,
        out_shape=(out, out, out),
    )(x, ws, bs, wd, bd, wres)


def _denser_body(h_ref, ws_ref, bs_ref, wd_ref, bd_ref, fs_o, fd_o):
    hb = h_ref[...]
    fs_o[...] = (jnp.dot(hb, ws_ref[...], preferred_element_type=_f32)
                 + bs_ref[...][None, :])
    fd_o[...] = (jnp.dot(hb, wd_ref[...], preferred_element_type=_f32)
                 + bd_ref[...][None, :])


def _denser(h, ws, bs, wd, bd):
    out = jax.ShapeDtypeStruct((N, NODES), _f32)
    return pl.pallas_call(
        _denser_body,
        grid=(GRID,),
        in_specs=[
            pl.BlockSpec((RB, NODES), lambda i: (i, 0)),
            pl.BlockSpec((NODES, NODES), lambda i: (0, 0)),
            pl.BlockSpec((NODES,), lambda i: (0,)),
            pl.BlockSpec((NODES, NODES), lambda i: (0, 0)),
            pl.BlockSpec((NODES,), lambda i: (0,)),
        ],
        out_specs=[pl.BlockSpec((RB, NODES), lambda i: (i, 0))] * 2,
        out_shape=(out, out),
    )(h, ws, bs, wd, bd)


def _node_body(out_ref, res_ref, g_ref, b_ref, h_o, feats_o, acc):
    i = pl.program_id(0)
    t = out_ref[...] + res_ref[...]

    @pl.when(i == 0)
    def _init():
        acc[...] = jnp.zeros_like(acc)

    @pl.when(i < GRID)
    def _accum():
        acc[0:1, :] = acc[0:1, :] + jnp.sum(t, axis=0, keepdims=True)
        acc[1:2, :] = acc[1:2, :] + jnp.sum(t * t, axis=0, keepdims=True)

    @pl.when(i == GRID - 1)
    def _stats():
        mu = acc[0:1, :] / float(N)
        var = acc[1:2, :] / float(N) - mu * mu
        acc[2:3, :] = mu
        acc[3:4, :] = lax.rsqrt(var + 1e-5)

    @pl.when(i >= GRID)
    def _norm():
        mu = acc[2:3, :]
        inv = acc[3:4, :]
        y = (t - mu) * inv * g_ref[...][None, :] + b_ref[...][None, :]
        y = jnp.maximum(y, 0.0)
        h_o[...] = y
        npg = N // B
        f0 = jnp.sum(y[0:npg, :], axis=0, keepdims=True) / float(npg)
        f1 = jnp.sum(y[npg:2 * npg, :], axis=0, keepdims=True) / float(npg)
        feats_o[...] = jnp.concatenate([f0, f1], axis=0)[None]


def _node(out_nodes, res, gamma, beta):
    blk = lambda i: (jnp.where(i < GRID, i, i - GRID), 0)
    return pl.pallas_call(
        _node_body,
        grid=(2 * GRID,),
        in_specs=[
            pl.BlockSpec((RB, NODES), blk),
            pl.BlockSpec((RB, NODES), blk),
            pl.BlockSpec((NODES,), lambda i: (0,)),
            pl.BlockSpec((NODES,), lambda i: (0,)),
        ],
        out_specs=[
            pl.BlockSpec((RB, NODES), blk),
            pl.BlockSpec((1, 2, NODES),
                         lambda i: (jnp.where(i < GRID, 0, i - GRID), 0, 0)),
        ],
        out_shape=(jax.ShapeDtypeStruct((N, NODES), _f32),
                   jax.ShapeDtypeStruct((GRID, 2, NODES), _f32)),
        scratch_shapes=[pltpu.VMEM((8, NODES), _f32)],
    )(out_nodes, res, gamma, beta)


def _head_body(feats_ref, w1_ref, b1_ref, gf_ref, bf_ref, w2_ref, b2_ref,
               out_ref):
    feats = feats_ref[...]
    f1 = jnp.dot(feats, w1_ref[...], preferred_element_type=_f32)
    f1 = f1 + b1_ref[...][None, :]
    mu = jnp.mean(f1, axis=0, keepdims=True)
    var = jnp.mean((f1 - mu) ** 2, axis=0, keepdims=True)
    f1 = (f1 - mu) * lax.rsqrt(var + 1e-5) * gf_ref[...][None, :]
    f1 = f1 + bf_ref[...][None, :]
    f1 = jnp.maximum(f1, 0.0)
    out = jnp.dot(f1, w2_ref[...], preferred_element_type=_f32)
    out_ref[...] = out + b2_ref[...][None, :]


def _head(feats, w1, b1, gf, bf, w2, b2):
    return pl.pallas_call(
        _head_body,
        out_shape=jax.ShapeDtypeStruct((B, 5), _f32),
    )(feats, w1, b1, gf, bf, w2, b2)


def _edge_phase(src, dst, fs, fd, avec):
    logits, mpart = _edge1(src, dst, fs, fd, avec)
    ex, denpart = _edge2(dst, logits, mpart)
    att_l = _alpha(dst, ex, denpart)
    outp = _edge3(src, dst, att_l, fs)
    out_nodes = jnp.concatenate([outp[0, :HALF], outp[1, :HALF]], axis=0)
    return att_l, out_nodes


def kernel(x, edge_index, ws0, bs0, wd0, bd0, a0, wres0, g0, be0, wsr, bsr,
           wdr, bdr, ar, gr, ber, w1, b1, gf, bf, w2, b2):
    src = edge_index[0]
    dst = edge_index[1]
    attns = []
    feats = []

    fs, fd, res = _dense0(x, ws0, bs0, wd0, bd0, wres0)
    att_l, out_nodes = _edge_phase(src, dst, fs, fd, a0.reshape(NODES))
    h, f_l = _node(out_nodes, res, g0, be0)
    attns.append(att_l)
    feats.append(f_l.reshape(B, NODES))

    for i in range(LAYERS - 1):
        fs, fd = _denser(h, wsr[i], bsr[i], wdr[i], bdr[i])
        att_l, out_nodes = _edge_phase(src, dst, fs, fd, ar[i].reshape(NODES))
        h_next, f_l = _node(out_nodes, h, gr[i], ber[i])
        h = h_next
        attns.append(att_l)
        feats.append(f_l.reshape(B, NODES))

    feat = jnp.concatenate(feats, axis=1)
    feat = _head(feat, w1, b1, gf, bf, w2, b2)
    att = jnp.stack(attns, axis=1)
    return feat, att


# speedup vs baseline: 6.8330x; 1.4050x over previous
import jax
import jax.numpy as jnp
from jax import lax
from jax.experimental import pallas as pl
from jax.experimental.pallas import tpu as pltpu
from jax.experimental.pallas import tpu_sc as plsc

N = 50000
E = 800000
B = 50
NODES = 64
HEADS = 1
DH = NODES // HEADS
IN0 = 21
LAYERS = 4

NC = 2
NS = 16
NW = NC * NS
L = 16

C = 128
NCH = E // C
NCHW = -(-NCH // NW)
NCHS = -(-NCH // NS)
C2 = 256
NCH2 = E // C2
NCH2W = -(-NCH2 // NW)
NPAD = 51200
SL = NPAD // NS
HALF = 25000
OUTROWS = 25600
SPROWS = 25600
TRASH = HALF
RB = 2000
GRID = N // RB

_f32 = jnp.float32
_i32 = jnp.int32

_SC_PARAMS = pltpu.CompilerParams(needs_layout_passes=False,
                                  use_tc_tiling_on_sc=False)


def _mesh():
    return plsc.VectorSubcoreMesh(core_axis_name="c", subcore_axis_name="s",
                                  num_cores=NC, num_subcores=NS)


def _wid():
    c = lax.axis_index("c")
    s = lax.axis_index("s")
    return c, s, s * NC + c


def _ring_reduce(s, arr, accred, tmpred, msh, combine):
    def cp16(dst_ref, src_vals_ref, off):
        def body(j, _):
            dst_ref[pl.ds(j * L, L)] = src_vals_ref[pl.ds(off + j * L, L)]
            return _
        lax.fori_loop(0, SL // L, body, None)

    cp16(accred, arr, s * SL)
    for r in range(1, NS):
        send_slice = lax.rem(s + r, NS)
        pltpu.sync_copy(arr.at[pl.ds(send_slice * SL, SL)], msh.at[s])
        plsc.subcore_barrier()
        read_row = lax.rem(s - r + NS, NS)
        pltpu.sync_copy(msh.at[read_row], tmpred)

        def mx(j, _):
            sl = pl.ds(j * L, L)
            accred[sl] = combine(accred[sl], tmpred[sl])
            return _
        lax.fori_loop(0, SL // L, mx, None)
        plsc.subcore_barrier()


def _edge1_body(ei, fs, fd, avec, logits_o, mpart_o,
                eb0, eb1, fsr0, fsr1, fdr0, fdr1, logbuf, tbuf, abuf_v,
                maxarr, accred, tmpred, msh, sem0, sem1):
    c, s, w = _wid()

    def ini(j, _):
        maxarr[pl.ds(j * L, L)] = jnp.full((L,), -3.4e38, _f32)
        return _
    lax.fori_loop(0, NPAD // L, ini, None)

    pltpu.sync_copy(avec, abuf_v)
    aq = [abuf_v[pl.ds(q * L, L)] for q in range(4)]
    iot16 = lax.iota(_i32, L) * L

    nch = jnp.where(w < NCH % NW, NCHW, NCH // NW)
    ebs = (eb0, eb1)
    fsrs = (fsr0, fsr1)
    fdrs = (fdr0, fdr1)
    sems = (sem0, sem1)

    def fetch(k, p):
        base = (w + k * NW) * C
        pltpu.sync_copy(ei.at[:, pl.ds(base, C)], ebs[p])
        pltpu.async_copy(fs.at[ebs[p].at[0]], fsrs[p], sems[p])
        pltpu.async_copy(fd.at[ebs[p].at[1]], fdrs[p], sems[p])

    def waitrows(p):
        pltpu.make_async_copy(fs.at[ebs[p].at[0]], fsrs[p], sems[p]).wait()
        pltpu.make_async_copy(fd.at[ebs[p].at[1]], fdrs[p], sems[p]).wait()

    def compute(k, p):
        base = (w + k * NW) * C
        fsr, fdr, eb = fsrs[p], fdrs[p], ebs[p]

        def grp(g, _):
            e0 = g * L
            for i in range(L):
                e = e0 + i
                acc = None
                for q in range(4):
                    t = fsr[e, pl.ds(q * L, L)] + fdr[e, pl.ds(q * L, L)]
                    lr = 0.2 * t + 0.8 * jnp.maximum(t, 0.0)
                    acc = lr * aq[q] if acc is None else acc + lr * aq[q]
                tbuf[pl.ds(i * L, L)] = acc
            lg = None
            for cc in range(L):
                v = plsc.load_gather(tbuf, [iot16 + cc])
                lg = v if lg is None else lg + v
            logbuf[pl.ds(g * L, L)] = lg
            dvec = eb[1, pl.ds(g * L, L)]
            old = plsc.load_gather(maxarr, [dvec])
            upd = lg > old
            plsc.store_scatter(maxarr, [dvec], jnp.maximum(old, lg), mask=upd)
            rb2 = plsc.load_gather(maxarr, [dvec])

            def w_body(q_):
                o2 = plsc.load_gather(maxarr, [dvec])
                u2 = lg > o2
                plsc.store_scatter(maxarr, [dvec], jnp.maximum(o2, lg),
                                   mask=u2)
                r2 = plsc.load_gather(maxarr, [dvec])
                return jnp.any(r2 < lg)
            lax.while_loop(lambda q_: q_, w_body, jnp.any(rb2 < lg))
            return _
        lax.fori_loop(0, C // L, grp, None)
        pltpu.sync_copy(logbuf, logits_o.at[pl.ds(base, C)])

    fetch(0, 0)

    def pair(kk, _):
        for p in range(2):
            k = kk * 2 + p

            @pl.when(k < nch)
            def _do(k=k, p=p):
                @pl.when(k + 1 < nch)
                def _pre():
                    fetch(k + 1, 1 - p)
                waitrows(p)
                compute(k, p)
        return _
    lax.fori_loop(0, (NCHW + 1) // 2, pair, None)

    plsc.subcore_barrier()
    _ring_reduce(s, maxarr, accred, tmpred, msh, jnp.maximum)
    pltpu.sync_copy(accred, mpart_o.at[c, pl.ds(s * SL, SL)])


_edge1 = pl.kernel(
    _edge1_body,
    out_type=(jax.ShapeDtypeStruct((E,), _f32),
              jax.ShapeDtypeStruct((NC, NPAD), _f32)),
    mesh=_mesh(),
    compiler_params=_SC_PARAMS,
    scratch_types=[
        pltpu.VMEM((2, C), _i32), pltpu.VMEM((2, C), _i32),
        pltpu.VMEM((C, NODES), _f32), pltpu.VMEM((C, NODES), _f32),
        pltpu.VMEM((C, NODES), _f32), pltpu.VMEM((C, NODES), _f32),
        pltpu.VMEM((C,), _f32), pltpu.VMEM((L * L,), _f32),
        pltpu.VMEM((NODES,), _f32),
        pltpu.VMEM((NPAD,), _f32),
        pltpu.VMEM((SL,), _f32), pltpu.VMEM((SL,), _f32),
        pltpu.VMEM_SHARED((NS, SL), _f32),
        pltpu.SemaphoreType.DMA, pltpu.SemaphoreType.DMA,
    ],
)


def _edge2_body(ei, logits, mpart, ex_o, denpart_o,
                eb, lbuf, exbuf, marr, denarr, accred, tmpred, msh):
    c, s, w = _wid()
    pltpu.sync_copy(mpart.at[0], marr)
    pltpu.sync_copy(mpart.at[1], denarr)

    def comb(j, _):
        sl = pl.ds(j * L, L)
        marr[sl] = jnp.maximum(marr[sl], denarr[sl])
        denarr[sl] = jnp.zeros((L,), _f32)
        return _
    lax.fori_loop(0, NPAD // L, comb, None)

    nch = jnp.where(w < NCH2 % NW, NCH2W, NCH2 // NW)

    def chunk(k, _):
        base = (w + k * NW) * C2
        pltpu.sync_copy(ei.at[:, pl.ds(base, C2)], eb)
        pltpu.sync_copy(logits.at[pl.ds(base, C2)], lbuf)

        def grp(g, _):
            dvec = eb[1, pl.ds(g * L, L)]
            lvec = lbuf[pl.ds(g * L, L)]
            mv = plsc.load_gather(marr, [dvec])
            exv = jnp.exp(lvec - mv)
            exbuf[pl.ds(g * L, L)] = exv
            plsc.addupdate_scatter(denarr, [dvec], exv)
            return _
        lax.fori_loop(0, C2 // L, grp, None)
        pltpu.sync_copy(exbuf, ex_o.at[pl.ds(base, C2)])
        return _
    lax.fori_loop(0, nch, chunk, None)

    plsc.subcore_barrier()
    _ring_reduce(s, denarr, accred, tmpred, msh, lambda a, b: a + b)
    pltpu.sync_copy(accred, denpart_o.at[c, pl.ds(s * SL, SL)])


_edge2 = pl.kernel(
    _edge2_body,
    out_type=(jax.ShapeDtypeStruct((E,), _f32),
              jax.ShapeDtypeStruct((NC, NPAD), _f32)),
    mesh=_mesh(),
    compiler_params=_SC_PARAMS,
    scratch_types=[
        pltpu.VMEM((2, C2), _i32), pltpu.VMEM((C2,), _f32),
        pltpu.VMEM((C2,), _f32),
        pltpu.VMEM((NPAD,), _f32), pltpu.VMEM((NPAD,), _f32),
        pltpu.VMEM((SL,), _f32), pltpu.VMEM((SL,), _f32),
        pltpu.VMEM_SHARED((NS, SL), _f32),
    ],
)


def _alpha_body(ei, ex, denpart, att_o,
                eb, exbuf, abuf, denarr, tmpd):
    c, s, w = _wid()
    pltpu.sync_copy(denpart.at[0], denarr)

    def comb_j(j, _):
        pltpu.sync_copy(denpart.at[1, pl.ds(j * SL, SL)], tmpd)

        def ad(t, _):
            denarr[pl.ds(j * SL + t * L, L)] = (
                denarr[pl.ds(j * SL + t * L, L)] + tmpd[pl.ds(t * L, L)])
            return _
        lax.fori_loop(0, SL // L, ad, None)
        return _
    lax.fori_loop(0, NS, comb_j, None)

    nch = jnp.where(w < NCH2 % NW, NCH2W, NCH2 // NW)

    def chunk(k, _):
        base = (w + k * NW) * C2
        pltpu.sync_copy(ei.at[:, pl.ds(base, C2)], eb)
        pltpu.sync_copy(ex.at[pl.ds(base, C2)], exbuf)

        def grp(g, _):
            dvec = eb[1, pl.ds(g * L, L)]
            exv = exbuf[pl.ds(g * L, L)]
            dv = plsc.load_gather(denarr, [dvec])
            abuf[pl.ds(g * L, L)] = exv / (dv + 1e-9)
            return _
        lax.fori_loop(0, C2 // L, grp, None)
        pltpu.sync_copy(abuf, att_o.at[pl.ds(base, C2)])
        return _
    lax.fori_loop(0, nch, chunk, None)


_alpha = pl.kernel(
    _alpha_body,
    out_type=jax.ShapeDtypeStruct((E,), _f32),
    mesh=_mesh(),
    compiler_params=_SC_PARAMS,
    scratch_types=[
        pltpu.VMEM((2, C2), _i32), pltpu.VMEM((C2,), _f32),
        pltpu.VMEM((C2,), _f32),
        pltpu.VMEM((NPAD,), _f32), pltpu.VMEM((SL,), _f32),
    ],
)


def _edge3_body(ei, att, fs, outp_o,
                eb0, eb1, ab0, ab1, fsr0, fsr1, rowbuf, locidx, spout,
                sem0, sem1):
    c, s, w = _wid()

    def zrow(i, _):
        for q in range(4):
            rowbuf[i, pl.ds(q * L, L)] = jnp.zeros((L,), _f32)
        return _
    lax.fori_loop(0, C, zrow, None)

    nz = SPROWS // C
    zc = jnp.where(s < nz % NS, nz // NS + 1, nz // NS)

    def zch(k, _):
        blk = s + k * NS
        pltpu.sync_copy(rowbuf, spout.at[pl.ds(blk * C, C)])
        return _
    lax.fori_loop(0, zc, zch, None)
    plsc.subcore_barrier()

    lo = c * HALF
    nch = jnp.where(s < NCH % NS, NCHS, NCH // NS)
    ebs = (eb0, eb1)
    abs_ = (ab0, ab1)
    fsrs = (fsr0, fsr1)
    sems = (sem0, sem1)

    def fetch(k, p):
        base = (s + k * NS) * C
        pltpu.sync_copy(ei.at[:, pl.ds(base, C)], ebs[p])
        pltpu.sync_copy(att.at[pl.ds(base, C)], abs_[p])
        pltpu.async_copy(fs.at[ebs[p].at[0]], fsrs[p], sems[p])

    def waitrows(p):
        pltpu.make_async_copy(fs.at[ebs[p].at[0]], fsrs[p], sems[p]).wait()

    def compute(k, p):
        eb, ab, fsr = ebs[p], abs_[p], fsrs[p]

        def grp(g, _):
            dvec = eb[1, pl.ds(g * L, L)]
            inh = (dvec >= lo) & (dvec < lo + HALF)
            locidx[pl.ds(g * L, L)] = jnp.where(inh, dvec - lo, TRASH)
            e0 = g * L
            for i in range(L):
                e = e0 + i
                av = plsc.load_gather(ab, [jnp.full((L,), e, _i32)])
                for q in range(4):
                    rowbuf[e, pl.ds(q * L, L)] = fsr[e, pl.ds(q * L, L)] * av
            return _
        lax.fori_loop(0, C // L, grp, None)
        pltpu.sync_copy(rowbuf, spout.at[locidx], add=True)

    fetch(0, 0)

    def pair(kk, _):
        for p in range(2):
            k = kk * 2 + p

            @pl.when(k < nch)
            def _do(k=k, p=p):
                @pl.when(k + 1 < nch)
                def _pre():
                    fetch(k + 1, 1 - p)
                waitrows(p)
                compute(k, p)
        return _
    lax.fori_loop(0, (NCHS + 1) // 2, pair, None)

    plsc.subcore_barrier()
    rows = OUTROWS // NS
    pltpu.sync_copy(spout.at[pl.ds(s * rows, rows)],
                    outp_o.at[c, pl.ds(s * rows, rows)])


_edge3 = pl.kernel(
    _edge3_body,
    out_type=jax.ShapeDtypeStruct((NC, OUTROWS, NODES), _f32),
    mesh=_mesh(),
    compiler_params=_SC_PARAMS,
    scratch_types=[
        pltpu.VMEM((2, C), _i32), pltpu.VMEM((2, C), _i32),
        pltpu.VMEM((C,), _f32), pltpu.VMEM((C,), _f32),
        pltpu.VMEM((C, NODES), _f32), pltpu.VMEM((C, NODES), _f32),
        pltpu.VMEM((C, NODES), _f32), pltpu.VMEM((C,), _i32),
        pltpu.VMEM_SHARED((SPROWS, NODES), _f32),
        pltpu.SemaphoreType.DMA, pltpu.SemaphoreType.DMA,
    ],
)


def _dense0_body(x_ref, ws_ref, bs_ref, wd_ref, bd_ref, wres_ref,
                 fs_o, fd_o, res_o):
    xb = x_ref[...]
    fs_o[...] = (jnp.dot(xb, ws_ref[...], preferred_element_type=_f32)
                 + bs_ref[...][None, :])
    fd_o[...] = (jnp.dot(xb, wd_ref[...], preferred_element_type=_f32)
                 + bd_ref[...][None, :])
    res_o[...] = jnp.dot(xb, wres_ref[...], preferred_element_type=_f32)


def _dense0(x, ws, bs, wd, bd, wres):
    out = jax.ShapeDtypeStruct((N, NODES), _f32)
    return pl.pallas_call(
        _dense0_body,
        grid=(GRID,),
        in_specs=[
            pl.BlockSpec((RB, IN0), lambda i: (i, 0)),
            pl.BlockSpec((IN0, NODES), lambda i: (0, 0)),
            pl.BlockSpec((NODES,), lambda i: (0,)),
            pl.BlockSpec((IN0, NODES), lambda i: (0, 0)),
            pl.BlockSpec((NODES,), lambda i: (0,)),
            pl.BlockSpec((IN0, NODES), lambda i: (0, 0)),
        ],
        out_specs=[pl.BlockSpec((RB, NODES), lambda i: (i, 0))] * 3,
        out_shape=(out, out, out),
    )(x, ws, bs, wd, bd, wres)


def _denser_body(h_ref, ws_ref, bs_ref, wd_ref, bd_ref, fs_o, fd_o):
    hb = h_ref[...]
    fs_o[...] = (jnp.dot(hb, ws_ref[...], preferred_element_type=_f32)
                 + bs_ref[...][None, :])
    fd_o[...] = (jnp.dot(hb, wd_ref[...], preferred_element_type=_f32)
                 + bd_ref[...][None, :])


def _denser(h, ws, bs, wd, bd):
    out = jax.ShapeDtypeStruct((N, NODES), _f32)
    return pl.pallas_call(
        _denser_body,
        grid=(GRID,),
        in_specs=[
            pl.BlockSpec((RB, NODES), lambda i: (i, 0)),
            pl.BlockSpec((NODES, NODES), lambda i: (0, 0)),
            pl.BlockSpec((NODES,), lambda i: (0,)),
            pl.BlockSpec((NODES, NODES), lambda i: (0, 0)),
            pl.BlockSpec((NODES,), lambda i: (0,)),
        ],
        out_specs=[pl.BlockSpec((RB, NODES), lambda i: (i, 0))] * 2,
        out_shape=(out, out),
    )(h, ws, bs, wd, bd)


def _node_body(out_ref, res_ref, g_ref, b_ref, h_o, feats_o, acc):
    i = pl.program_id(0)
    t = out_ref[...] + res_ref[...]

    @pl.when(i == 0)
    def _init():
        acc[...] = jnp.zeros_like(acc)

    @pl.when(i < GRID)
    def _accum():
        acc[0:1, :] = acc[0:1, :] + jnp.sum(t, axis=0, keepdims=True)
        acc[1:2, :] = acc[1:2, :] + jnp.sum(t * t, axis=0, keepdims=True)

    @pl.when(i == GRID - 1)
    def _stats():
        mu = acc[0:1, :] / float(N)
        var = acc[1:2, :] / float(N) - mu * mu
        acc[2:3, :] = mu
        acc[3:4, :] = lax.rsqrt(var + 1e-5)

    @pl.when(i >= GRID)
    def _norm():
        mu = acc[2:3, :]
        inv = acc[3:4, :]
        y = (t - mu) * inv * g_ref[...][None, :] + b_ref[...][None, :]
        y = jnp.maximum(y, 0.0)
        h_o[...] = y
        npg = N // B
        f0 = jnp.sum(y[0:npg, :], axis=0, keepdims=True) / float(npg)
        f1 = jnp.sum(y[npg:2 * npg, :], axis=0, keepdims=True) / float(npg)
        feats_o[...] = jnp.concatenate([f0, f1], axis=0)[None]


def _node(out_nodes, res, gamma, beta):
    blk = lambda i: (jnp.where(i < GRID, i, i - GRID), 0)
    return pl.pallas_call(
        _node_body,
        grid=(2 * GRID,),
        in_specs=[
            pl.BlockSpec((RB, NODES), blk),
            pl.BlockSpec((RB, NODES), blk),
            pl.BlockSpec((NODES,), lambda i: (0,)),
            pl.BlockSpec((NODES,), lambda i: (0,)),
        ],
        out_specs=[
            pl.BlockSpec((RB, NODES), blk),
            pl.BlockSpec((1, 2, NODES),
                         lambda i: (jnp.where(i < GRID, 0, i - GRID), 0, 0)),
        ],
        out_shape=(jax.ShapeDtypeStruct((N, NODES), _f32),
                   jax.ShapeDtypeStruct((GRID, 2, NODES), _f32)),
        scratch_shapes=[pltpu.VMEM((8, NODES), _f32)],
    )(out_nodes, res, gamma, beta)


def _head_body(feats_ref, w1_ref, b1_ref, gf_ref, bf_ref, w2_ref, b2_ref,
               out_ref):
    feats = feats_ref[...]
    f1 = jnp.dot(feats, w1_ref[...], preferred_element_type=_f32)
    f1 = f1 + b1_ref[...][None, :]
    mu = jnp.mean(f1, axis=0, keepdims=True)
    var = jnp.mean((f1 - mu) ** 2, axis=0, keepdims=True)
    f1 = (f1 - mu) * lax.rsqrt(var + 1e-5) * gf_ref[...][None, :]
    f1 = f1 + bf_ref[...][None, :]
    f1 = jnp.maximum(f1, 0.0)
    out = jnp.dot(f1, w2_ref[...], preferred_element_type=_f32)
    out_ref[...] = out + b2_ref[...][None, :]


def _head(feats, w1, b1, gf, bf, w2, b2):
    return pl.pallas_call(
        _head_body,
        out_shape=jax.ShapeDtypeStruct((B, 5), _f32),
    )(feats, w1, b1, gf, bf, w2, b2)


def _edge_phase(ei, fs, fd, avec):
    logits, mpart = _edge1(ei, fs, fd, avec)
    ex, denpart = _edge2(ei, logits, mpart)
    att_l = _alpha(ei, ex, denpart)
    outp = _edge3(ei, att_l, fs)
    out_nodes = jnp.concatenate([outp[0, :HALF], outp[1, :HALF]], axis=0)
    return att_l, out_nodes


def kernel(x, edge_index, ws0, bs0, wd0, bd0, a0, wres0, g0, be0, wsr, bsr,
           wdr, bdr, ar, gr, ber, w1, b1, gf, bf, w2, b2):
    ei = edge_index
    attns = []
    feats = []

    fs, fd, res = _dense0(x, ws0, bs0, wd0, bd0, wres0)
    att_l, out_nodes = _edge_phase(ei, fs, fd, a0.reshape(NODES))
    h, f_l = _node(out_nodes, res, g0, be0)
    attns.append(att_l)
    feats.append(f_l.reshape(B, NODES))

    for i in range(LAYERS - 1):
        fs, fd = _denser(h, wsr[i], bsr[i], wdr[i], bdr[i])
        att_l, out_nodes = _edge_phase(ei, fs, fd, ar[i].reshape(NODES))
        h_next, f_l = _node(out_nodes, h, gr[i], ber[i])
        h = h_next
        attns.append(att_l)
        feats.append(f_l.reshape(B, NODES))

    feat = jnp.concatenate(feats, axis=1)
    feat = _head(feat, w1, b1, gf, bf, w2, b2)
    att = jnp.stack(attns, axis=1)
    return feat, att


# speedup vs baseline: 7.3323x; 1.0731x over previous
import jax
import jax.numpy as jnp
from jax import lax
from jax.experimental import pallas as pl
from jax.experimental.pallas import tpu as pltpu
from jax.experimental.pallas import tpu_sc as plsc

N = 50000
E = 800000
B = 50
NODES = 64
HEADS = 1
DH = NODES // HEADS
IN0 = 21
LAYERS = 4

NC = 2
NS = 16
NW = NC * NS
L = 16

C = 128
NCH = E // C
NCHW = -(-NCH // NW)
NCHS = -(-NCH // NS)
C2 = 256
NCH2 = E // C2
NCH2W = -(-NCH2 // NW)
NPAD = 51200
SL = NPAD // NS
HALF = 25000
OUTROWS = 25600
SPROWS = 25600
TRASH = HALF
RB = 2000
GRID = N // RB

_f32 = jnp.float32
_i32 = jnp.int32

_SC_PARAMS = pltpu.CompilerParams(needs_layout_passes=False,
                                  use_tc_tiling_on_sc=False)


def _mesh():
    return plsc.VectorSubcoreMesh(core_axis_name="c", subcore_axis_name="s",
                                  num_cores=NC, num_subcores=NS)


def _wid():
    c = lax.axis_index("c")
    s = lax.axis_index("s")
    return c, s, s * NC + c


def _ring_reduce(s, arr, accred, tmpred, msh, combine):
    def cp16(dst_ref, src_vals_ref, off):
        def body(j, _):
            dst_ref[pl.ds(j * L, L)] = src_vals_ref[pl.ds(off + j * L, L)]
            return _
        lax.fori_loop(0, SL // L, body, None)

    cp16(accred, arr, s * SL)
    for r in range(1, NS):
        send_slice = lax.rem(s + r, NS)
        pltpu.sync_copy(arr.at[pl.ds(send_slice * SL, SL)], msh.at[s])
        plsc.subcore_barrier()
        read_row = lax.rem(s - r + NS, NS)
        pltpu.sync_copy(msh.at[read_row], tmpred)

        def mx(j, _):
            sl = pl.ds(j * L, L)
            accred[sl] = combine(accred[sl], tmpred[sl])
            return _
        lax.fori_loop(0, SL // L, mx, None)
        plsc.subcore_barrier()


def _edge1_body(ei, fs, fd, avec, logits_o, mpart_o,
                eb0, eb1, fsr0, fsr1, fdr0, fdr1, lb0, lb1, tbuf, abuf_v,
                maxarr, accred, tmpred, msh, sem0, sem1, semL0, semL1):
    c, s, w = _wid()

    def ini(j, _):
        maxarr[pl.ds(j * L, L)] = jnp.full((L,), -3.4e38, _f32)
        return _
    lax.fori_loop(0, NPAD // L, ini, None)

    pltpu.sync_copy(avec, abuf_v)
    aq = [abuf_v[pl.ds(q * L, L)] for q in range(4)]
    iot17 = lax.iota(_i32, L) * (L + 1)

    nch = jnp.where(w < NCH % NW, NCHW, NCH // NW)
    ebs = (eb0, eb1)
    fsrs = (fsr0, fsr1)
    fdrs = (fdr0, fdr1)
    sems = (sem0, sem1)
    lbs = (lb0, lb1)
    semLs = (semL0, semL1)

    def fetch(k, p):
        base = (w + k * NW) * C
        pltpu.sync_copy(ei.at[:, pl.ds(base, C)], ebs[p])
        pltpu.async_copy(fs.at[ebs[p].at[0]], fsrs[p], sems[p])
        pltpu.async_copy(fd.at[ebs[p].at[1]], fdrs[p], sems[p])

    def waitrows(p):
        pltpu.make_async_copy(fs.at[ebs[p].at[0]], fsrs[p], sems[p]).wait()
        pltpu.make_async_copy(fd.at[ebs[p].at[1]], fdrs[p], sems[p]).wait()

    def compute(k, p):
        base = (w + k * NW) * C
        fsr, fdr, eb = fsrs[p], fdrs[p], ebs[p]
        logbuf = lbs[p]

        @pl.when(k >= 2)
        def _drain():
            pltpu.make_async_copy(logbuf, logits_o.at[pl.ds(base, C)],
                                  semLs[p]).wait()

        def grp(g, _):
            e0 = g * L
            for i in range(L):
                e = e0 + i
                acc = None
                for q in range(4):
                    t = fsr[e, pl.ds(q * L, L)] + fdr[e, pl.ds(q * L, L)]
                    lr = 0.2 * t + 0.8 * jnp.maximum(t, 0.0)
                    acc = lr * aq[q] if acc is None else acc + lr * aq[q]
                tbuf[pl.ds(i * (L + 1), L)] = acc
            lg = None
            for cc in range(L):
                v = plsc.load_gather(tbuf, [iot17 + cc])
                lg = v if lg is None else lg + v
            logbuf[pl.ds(g * L, L)] = lg
            dvec = eb[1, pl.ds(g * L, L)]
            old = plsc.load_gather(maxarr, [dvec])
            upd = lg > old
            plsc.store_scatter(maxarr, [dvec], jnp.maximum(old, lg), mask=upd)
            rb2 = plsc.load_gather(maxarr, [dvec])

            def w_body(q_):
                o2 = plsc.load_gather(maxarr, [dvec])
                u2 = lg > o2
                plsc.store_scatter(maxarr, [dvec], jnp.maximum(o2, lg),
                                   mask=u2)
                r2 = plsc.load_gather(maxarr, [dvec])
                return jnp.any(r2 < lg)
            lax.while_loop(lambda q_: q_, w_body, jnp.any(rb2 < lg))
            return _
        lax.fori_loop(0, C // L, grp, None)
        pltpu.async_copy(logbuf, logits_o.at[pl.ds(base, C)], semLs[p])

    fetch(0, 0)

    def pair(kk, _):
        for p in range(2):
            k = kk * 2 + p

            @pl.when(k < nch)
            def _do(k=k, p=p):
                @pl.when(k + 1 < nch)
                def _pre():
                    fetch(k + 1, 1 - p)
                waitrows(p)
                compute(k, p)
        return _
    lax.fori_loop(0, (NCHW + 1) // 2, pair, None)
    for p in range(2):
        pltpu.make_async_copy(lbs[p], logits_o.at[pl.ds(0, C)],
                              semLs[p]).wait()

    plsc.subcore_barrier()
    _ring_reduce(s, maxarr, accred, tmpred, msh, jnp.maximum)
    pltpu.sync_copy(accred, mpart_o.at[c, pl.ds(s * SL, SL)])


_edge1 = pl.kernel(
    _edge1_body,
    out_type=(jax.ShapeDtypeStruct((E,), _f32),
              jax.ShapeDtypeStruct((NC, NPAD), _f32)),
    mesh=_mesh(),
    compiler_params=_SC_PARAMS,
    scratch_types=[
        pltpu.VMEM((2, C), _i32), pltpu.VMEM((2, C), _i32),
        pltpu.VMEM((C, NODES), _f32), pltpu.VMEM((C, NODES), _f32),
        pltpu.VMEM((C, NODES), _f32), pltpu.VMEM((C, NODES), _f32),
        pltpu.VMEM((C,), _f32), pltpu.VMEM((C,), _f32),
        pltpu.VMEM((L * (L + 1),), _f32),
        pltpu.VMEM((NODES,), _f32),
        pltpu.VMEM((NPAD,), _f32),
        pltpu.VMEM((SL,), _f32), pltpu.VMEM((SL,), _f32),
        pltpu.VMEM_SHARED((NS, SL), _f32),
        pltpu.SemaphoreType.DMA, pltpu.SemaphoreType.DMA,
        pltpu.SemaphoreType.DMA, pltpu.SemaphoreType.DMA,
    ],
)


def _edge2_body(ei, logits, mpart, ex_o, denpart_o,
                eb, lbuf, exbuf, marr, denarr, accred, tmpred, msh):
    c, s, w = _wid()
    pltpu.sync_copy(mpart.at[0], marr)
    pltpu.sync_copy(mpart.at[1], denarr)

    def comb(j, _):
        sl = pl.ds(j * L, L)
        marr[sl] = jnp.maximum(marr[sl], denarr[sl])
        denarr[sl] = jnp.zeros((L,), _f32)
        return _
    lax.fori_loop(0, NPAD // L, comb, None)

    nch = jnp.where(w < NCH2 % NW, NCH2W, NCH2 // NW)

    def chunk(k, _):
        base = (w + k * NW) * C2
        pltpu.sync_copy(ei.at[:, pl.ds(base, C2)], eb)
        pltpu.sync_copy(logits.at[pl.ds(base, C2)], lbuf)

        def grp(g, _):
            dvec = eb[1, pl.ds(g * L, L)]
            lvec = lbuf[pl.ds(g * L, L)]
            mv = plsc.load_gather(marr, [dvec])
            exv = jnp.exp(lvec - mv)
            exbuf[pl.ds(g * L, L)] = exv
            plsc.addupdate_scatter(denarr, [dvec], exv)
            return _
        lax.fori_loop(0, C2 // L, grp, None)
        pltpu.sync_copy(exbuf, ex_o.at[pl.ds(base, C2)])
        return _
    lax.fori_loop(0, nch, chunk, None)

    plsc.subcore_barrier()
    _ring_reduce(s, denarr, accred, tmpred, msh, lambda a, b: a + b)
    pltpu.sync_copy(accred, denpart_o.at[c, pl.ds(s * SL, SL)])


_edge2 = pl.kernel(
    _edge2_body,
    out_type=(jax.ShapeDtypeStruct((E,), _f32),
              jax.ShapeDtypeStruct((NC, NPAD), _f32)),
    mesh=_mesh(),
    compiler_params=_SC_PARAMS,
    scratch_types=[
        pltpu.VMEM((2, C2), _i32), pltpu.VMEM((C2,), _f32),
        pltpu.VMEM((C2,), _f32),
        pltpu.VMEM((NPAD,), _f32), pltpu.VMEM((NPAD,), _f32),
        pltpu.VMEM((SL,), _f32), pltpu.VMEM((SL,), _f32),
        pltpu.VMEM_SHARED((NS, SL), _f32),
    ],
)


def _edge3_body(ei, ex, den, fs, att_o, outp_o,
                eb0, eb1, xb0, xb1, dr0, dr1, ab0, ab1, fsr0, fsr1,
                rowbuf, locidx, spout, sem0, sem1, semA0, semA1):
    c, s, w = _wid()

    def zrow(i, _):
        for q in range(4):
            rowbuf[i, pl.ds(q * L, L)] = jnp.zeros((L,), _f32)
        return _
    lax.fori_loop(0, C, zrow, None)

    nz = SPROWS // C
    zc = jnp.where(s < nz % NS, nz // NS + 1, nz // NS)

    def zch(k, _):
        blk = s + k * NS
        pltpu.sync_copy(rowbuf, spout.at[pl.ds(blk * C, C)])
        return _
    lax.fori_loop(0, zc, zch, None)
    plsc.subcore_barrier()

    lo = c * HALF
    nch = jnp.where(s < NCH % NS, NCHS, NCH // NS)
    ebs = (eb0, eb1)
    xbs = (xb0, xb1)
    drs = (dr0, dr1)
    abs_ = (ab0, ab1)
    fsrs = (fsr0, fsr1)
    sems = (sem0, sem1)
    semAs = (semA0, semA1)

    def fetch(k, p):
        base = (s + k * NS) * C
        pltpu.sync_copy(ei.at[:, pl.ds(base, C)], ebs[p])
        pltpu.sync_copy(ex.at[pl.ds(base, C)], xbs[p])
        pltpu.async_copy(fs.at[ebs[p].at[0]], fsrs[p], sems[p])
        pltpu.async_copy(den.at[ebs[p].at[1]], drs[p], sems[p])

    def waitrows(p):
        pltpu.make_async_copy(fs.at[ebs[p].at[0]], fsrs[p], sems[p]).wait()
        pltpu.make_async_copy(den.at[ebs[p].at[1]], drs[p], sems[p]).wait()

    def compute(k, p):
        base = (s + k * NS) * C
        eb, xb, dr, ab, fsr = ebs[p], xbs[p], drs[p], abs_[p], fsrs[p]

        @pl.when((k >= 2) & (c == 0))
        def _drain():
            pltpu.make_async_copy(ab, att_o.at[pl.ds(base, C)],
                                  semAs[p]).wait()

        def grp(g, _):
            sl = pl.ds(g * L, L)
            dvec = eb[1, sl]
            ab[sl] = xb[sl] / (dr[sl] + 1e-9)
            inh = (dvec >= lo) & (dvec < lo + HALF)
            locidx[sl] = jnp.where(inh, dvec - lo, TRASH)
            e0 = g * L
            for i in range(L):
                e = e0 + i
                av = plsc.load_gather(ab, [jnp.full((L,), e, _i32)])
                for q in range(4):
                    rowbuf[e, pl.ds(q * L, L)] = fsr[e, pl.ds(q * L, L)] * av
            return _
        lax.fori_loop(0, C // L, grp, None)

        @pl.when(c == 0)
        def _st():
            pltpu.async_copy(ab, att_o.at[pl.ds(base, C)], semAs[p])
        pltpu.sync_copy(rowbuf, spout.at[locidx], add=True)

    fetch(0, 0)

    def pair(kk, _):
        for p in range(2):
            k = kk * 2 + p

            @pl.when(k < nch)
            def _do(k=k, p=p):
                @pl.when(k + 1 < nch)
                def _pre():
                    fetch(k + 1, 1 - p)
                waitrows(p)
                compute(k, p)
        return _
    lax.fori_loop(0, (NCHS + 1) // 2, pair, None)

    @pl.when(c == 0)
    def _fin():
        for p in range(2):
            pltpu.make_async_copy(abs_[p], att_o.at[pl.ds(0, C)],
                                  semAs[p]).wait()

    plsc.subcore_barrier()
    rows = OUTROWS // NS
    pltpu.sync_copy(spout.at[pl.ds(s * rows, rows)],
                    outp_o.at[c, pl.ds(s * rows, rows)])


_edge3 = pl.kernel(
    _edge3_body,
    out_type=(jax.ShapeDtypeStruct((E,), _f32),
              jax.ShapeDtypeStruct((NC, OUTROWS, NODES), _f32)),
    mesh=_mesh(),
    compiler_params=_SC_PARAMS,
    scratch_types=[
        pltpu.VMEM((2, C), _i32), pltpu.VMEM((2, C), _i32),
        pltpu.VMEM((C,), _f32), pltpu.VMEM((C,), _f32),
        pltpu.VMEM((C,), _f32), pltpu.VMEM((C,), _f32),
        pltpu.VMEM((C,), _f32), pltpu.VMEM((C,), _f32),
        pltpu.VMEM((C, NODES), _f32), pltpu.VMEM((C, NODES), _f32),
        pltpu.VMEM((C, NODES), _f32), pltpu.VMEM((C,), _i32),
        pltpu.VMEM_SHARED((SPROWS, NODES), _f32),
        pltpu.SemaphoreType.DMA, pltpu.SemaphoreType.DMA,
        pltpu.SemaphoreType.DMA, pltpu.SemaphoreType.DMA,
    ],
)


def _combine_body(dp_ref, o_ref):
    o_ref[...] = dp_ref[0, :] + dp_ref[1, :]


def _combine(denpart):
    return pl.pallas_call(
        _combine_body,
        out_shape=jax.ShapeDtypeStruct((NPAD,), _f32),
    )(denpart)


def _dense0_body(x_ref, ws_ref, bs_ref, wd_ref, bd_ref, wres_ref,
                 fs_o, fd_o, res_o):
    xb = x_ref[...]
    fs_o[...] = (jnp.dot(xb, ws_ref[...], preferred_element_type=_f32)
                 + bs_ref[...][None, :])
    fd_o[...] = (jnp.dot(xb, wd_ref[...], preferred_element_type=_f32)
                 + bd_ref[...][None, :])
    res_o[...] = jnp.dot(xb, wres_ref[...], preferred_element_type=_f32)


def _dense0(x, ws, bs, wd, bd, wres):
    out = jax.ShapeDtypeStruct((N, NODES), _f32)
    return pl.pallas_call(
        _dense0_body,
        grid=(GRID,),
        in_specs=[
            pl.BlockSpec((RB, IN0), lambda i: (i, 0)),
            pl.BlockSpec((IN0, NODES), lambda i: (0, 0)),
            pl.BlockSpec((NODES,), lambda i: (0,)),
            pl.BlockSpec((IN0, NODES), lambda i: (0, 0)),
            pl.BlockSpec((NODES,), lambda i: (0,)),
            pl.BlockSpec((IN0, NODES), lambda i: (0, 0)),
        ],
        out_specs=[pl.BlockSpec((RB, NODES), lambda i: (i, 0))] * 3,
        out_shape=(out, out, out),
    )(x, ws, bs, wd, bd, wres)


def _denser_body(h_ref, ws_ref, bs_ref, wd_ref, bd_ref, fs_o, fd_o):
    hb = h_ref[...]
    fs_o[...] = (jnp.dot(hb, ws_ref[...], preferred_element_type=_f32)
                 + bs_ref[...][None, :])
    fd_o[...] = (jnp.dot(hb, wd_ref[...], preferred_element_type=_f32)
                 + bd_ref[...][None, :])


def _denser(h, ws, bs, wd, bd):
    out = jax.ShapeDtypeStruct((N, NODES), _f32)
    return pl.pallas_call(
        _denser_body,
        grid=(GRID,),
        in_specs=[
            pl.BlockSpec((RB, NODES), lambda i: (i, 0)),
            pl.BlockSpec((NODES, NODES), lambda i: (0, 0)),
            pl.BlockSpec((NODES,), lambda i: (0,)),
            pl.BlockSpec((NODES, NODES), lambda i: (0, 0)),
            pl.BlockSpec((NODES,), lambda i: (0,)),
        ],
        out_specs=[pl.BlockSpec((RB, NODES), lambda i: (i, 0))] * 2,
        out_shape=(out, out),
    )(h, ws, bs, wd, bd)


def _node_body(out_ref, res_ref, g_ref, b_ref, h_o, feats_o, acc):
    i = pl.program_id(0)
    t = out_ref[...] + res_ref[...]

    @pl.when(i == 0)
    def _init():
        acc[...] = jnp.zeros_like(acc)

    @pl.when(i < GRID)
    def _accum():
        acc[0:1, :] = acc[0:1, :] + jnp.sum(t, axis=0, keepdims=True)
        acc[1:2, :] = acc[1:2, :] + jnp.sum(t * t, axis=0, keepdims=True)

    @pl.when(i == GRID - 1)
    def _stats():
        mu = acc[0:1, :] / float(N)
        var = acc[1:2, :] / float(N) - mu * mu
        acc[2:3, :] = mu
        acc[3:4, :] = lax.rsqrt(var + 1e-5)

    @pl.when(i >= GRID)
    def _norm():
        mu = acc[2:3, :]
        inv = acc[3:4, :]
        y = (t - mu) * inv * g_ref[...][None, :] + b_ref[...][None, :]
        y = jnp.maximum(y, 0.0)
        h_o[...] = y
        npg = N // B
        f0 = jnp.sum(y[0:npg, :], axis=0, keepdims=True) / float(npg)
        f1 = jnp.sum(y[npg:2 * npg, :], axis=0, keepdims=True) / float(npg)
        feats_o[...] = jnp.concatenate([f0, f1], axis=0)[None]


def _node(out_nodes, res, gamma, beta):
    blk = lambda i: (jnp.where(i < GRID, i, i - GRID), 0)
    return pl.pallas_call(
        _node_body,
        grid=(2 * GRID,),
        in_specs=[
            pl.BlockSpec((RB, NODES), blk),
            pl.BlockSpec((RB, NODES), blk),
            pl.BlockSpec((NODES,), lambda i: (0,)),
            pl.BlockSpec((NODES,), lambda i: (0,)),
        ],
        out_specs=[
            pl.BlockSpec((RB, NODES), blk),
            pl.BlockSpec((1, 2, NODES),
                         lambda i: (jnp.where(i < GRID, 0, i - GRID), 0, 0)),
        ],
        out_shape=(jax.ShapeDtypeStruct((N, NODES), _f32),
                   jax.ShapeDtypeStruct((GRID, 2, NODES), _f32)),
        scratch_shapes=[pltpu.VMEM((8, NODES), _f32)],
    )(out_nodes, res, gamma, beta)


def _head_body(feats_ref, w1_ref, b1_ref, gf_ref, bf_ref, w2_ref, b2_ref,
               out_ref):
    feats = feats_ref[...]
    f1 = jnp.dot(feats, w1_ref[...], preferred_element_type=_f32)
    f1 = f1 + b1_ref[...][None, :]
    mu = jnp.mean(f1, axis=0, keepdims=True)
    var = jnp.mean((f1 - mu) ** 2, axis=0, keepdims=True)
    f1 = (f1 - mu) * lax.rsqrt(var + 1e-5) * gf_ref[...][None, :]
    f1 = f1 + bf_ref[...][None, :]
    f1 = jnp.maximum(f1, 0.0)
    out = jnp.dot(f1, w2_ref[...], preferred_element_type=_f32)
    out_ref[...] = out + b2_ref[...][None, :]


def _head(feats, w1, b1, gf, bf, w2, b2):
    return pl.pallas_call(
        _head_body,
        out_shape=jax.ShapeDtypeStruct((B, 5), _f32),
    )(feats, w1, b1, gf, bf, w2, b2)


def _edge_phase(ei, fs, fd, avec):
    logits, mpart = _edge1(ei, fs, fd, avec)
    ex, denpart = _edge2(ei, logits, mpart)
    den = _combine(denpart)
    att_l, outp = _edge3(ei, ex, den, fs)
    out_nodes = jnp.concatenate([outp[0, :HALF], outp[1, :HALF]], axis=0)
    return att_l, out_nodes


def kernel(x, edge_index, ws0, bs0, wd0, bd0, a0, wres0, g0, be0, wsr, bsr,
           wdr, bdr, ar, gr, ber, w1, b1, gf, bf, w2, b2):
    ei = edge_index
    attns = []
    feats = []

    fs, fd, res = _dense0(x, ws0, bs0, wd0, bd0, wres0)
    att_l, out_nodes = _edge_phase(ei, fs, fd, a0.reshape(NODES))
    h, f_l = _node(out_nodes, res, g0, be0)
    attns.append(att_l)
    feats.append(f_l.reshape(B, NODES))

    for i in range(LAYERS - 1):
        fs, fd = _denser(h, wsr[i], bsr[i], wdr[i], bdr[i])
        att_l, out_nodes = _edge_phase(ei, fs, fd, ar[i].reshape(NODES))
        h_next, f_l = _node(out_nodes, h, gr[i], ber[i])
        h = h_next
        attns.append(att_l)
        feats.append(f_l.reshape(B, NODES))

    feat = jnp.concatenate(feats, axis=1)
    feat = _head(feat, w1, b1, gf, bf, w2, b2)
    att = jnp.stack(attns, axis=1)
    return feat, att
